# Initial kernel scaffold; baseline (speedup 1.0000x reference)
import jax
import jax.numpy as jnp
from jax.experimental import pallas as pl


def kernel(discrete_expression, normalized_expr, bin_table, W, b):
    raise NotImplementedError("write your pallas kernel here")


# speedup vs baseline: 2.0917x; 2.0917x over previous
import functools

import jax
import jax.numpy as jnp
from jax import lax
from jax.experimental import pallas as pl
from jax.experimental.pallas import tpu as pltpu
from jax.experimental.pallas import tpu_sc as plsc

EMBED_DIM = 64
LANES = 16
NUM_CORES = 2
NUM_SUBCORES = 16
NUM_WORKERS = NUM_CORES * NUM_SUBCORES
CHUNK = 1024
GATHER = 128
SLICES = EMBED_DIM // LANES


def _body(idx_hbm, x_hbm, tab_hbm, w_hbm, b_hbm, out_hbm,
          idx_v, x_v, rows_v, wb_v, sem):
    wid = lax.axis_index("s") * NUM_CORES + lax.axis_index("c")
    rows_per_worker = out_hbm.shape[0] // NUM_WORKERS
    n_chunks = rows_per_worker // CHUNK

    pltpu.sync_copy(w_hbm, wb_v.at[0])
    pltpu.sync_copy(b_hbm, wb_v.at[1])
    w_regs = [wb_v[0, pl.ds(c * LANES, LANES)] for c in range(SLICES)]
    b_regs = [wb_v[1, pl.ds(c * LANES, LANES)] for c in range(SLICES)]

    def chunk_body(ci, _):
        base = pl.multiple_of(wid * rows_per_worker + ci * CHUNK, CHUNK)
        pltpu.sync_copy(idx_hbm.at[pl.ds(pl.multiple_of(base // GATHER, 8),
                                         CHUNK // GATHER)],
                        idx_v)
        pltpu.sync_copy(x_hbm.at[pl.ds(base, CHUNK)], x_v)
        copies = [
            pltpu.async_copy(tab_hbm.at[idx_v.at[j]],
                             rows_v.at[pl.ds(j * GATHER, GATHER)], sem)
            for j in range(CHUNK // GATHER)
        ]
        for cp in copies:
            cp.wait()

        def row_body(r, _):
            xs = plsc.load_gather(x_v, [jnp.broadcast_to(r, (LANES,))])
            for c in range(SLICES):
                sl = pl.ds(c * LANES, LANES)
                rows_v[r, sl] = rows_v[r, sl] + xs * w_regs[c] + b_regs[c]
            return _

        lax.fori_loop(0, CHUNK, row_body, None)
        pltpu.sync_copy(rows_v, out_hbm.at[pl.ds(base, CHUNK)])
        return _

    lax.fori_loop(0, n_chunks, chunk_body, None)


def kernel(discrete_expression, normalized_expr, bin_table, W, b):
    B, G = discrete_expression.shape
    N = B * G
    idx = discrete_expression.astype(jnp.int32).reshape(N // GATHER, GATHER)
    x = normalized_expr.reshape(N)
    w = W[:, 0]

    mesh = plsc.VectorSubcoreMesh(core_axis_name="c", subcore_axis_name="s")
    run = pl.kernel(
        _body,
        out_type=jax.ShapeDtypeStruct((N, EMBED_DIM), jnp.float32),
        mesh=mesh,
        compiler_params=pltpu.CompilerParams(
            needs_layout_passes=False, use_tc_tiling_on_sc=False),
        scratch_types=[
            pltpu.VMEM((CHUNK // GATHER, GATHER), jnp.int32),
            pltpu.VMEM((CHUNK,), jnp.float32),
            pltpu.VMEM((CHUNK, EMBED_DIM), jnp.float32),
            pltpu.VMEM((2, EMBED_DIM), jnp.float32),
            pltpu.SemaphoreType.DMA,
        ],
    )
    out = run(idx, x, bin_table, w, b)
    return out.reshape(B, G, EMBED_DIM)


# speedup vs baseline: 2.3949x; 1.1450x over previous
import jax
import jax.numpy as jnp
from jax import lax
from jax.experimental import pallas as pl
from jax.experimental.pallas import tpu as pltpu
from jax.experimental.pallas import tpu_sc as plsc

EMBED_DIM = 64
LANES = 16
NUM_CORES = 2
NUM_SUBCORES = 16
NUM_WORKERS = NUM_CORES * NUM_SUBCORES
CHUNK = 512
SLICES = EMBED_DIM // LANES
VOCAB = 53


def _body(idx_hbm, x_hbm, tab_hbm, w_hbm, b_hbm, out_hbm,
          tab_v, idx_v, x_v, rows_v, wb_v, in_sems, out_sems):
    wid = lax.axis_index("s") * NUM_CORES + lax.axis_index("c")
    rows_per_worker = out_hbm.shape[0] // NUM_WORKERS
    n_chunks = rows_per_worker // CHUNK
    worker_base = wid * rows_per_worker

    pltpu.sync_copy(w_hbm, wb_v.at[0])
    pltpu.sync_copy(b_hbm, wb_v.at[1])
    pltpu.sync_copy(tab_hbm, tab_v)
    w_regs = [wb_v[0, pl.ds(c * LANES, LANES)] for c in range(SLICES)]
    b_regs = [wb_v[1, pl.ds(c * LANES, LANES)] for c in range(SLICES)]

    def fold_row(v, _):
        for c in range(SLICES):
            sl = pl.ds(c * LANES, LANES)
            tab_v[v, sl] = tab_v[v, sl] + b_regs[c]
        return _

    lax.fori_loop(0, VOCAB, fold_row, None)

    col_regs = [c * LANES + lax.iota(jnp.int32, LANES) for c in range(SLICES)]

    def fetch(ci, buf):
        base = worker_base + ci * CHUNK
        pltpu.async_copy(idx_hbm.at[pl.ds(base, CHUNK)], idx_v.at[buf],
                         in_sems.at[buf])
        pltpu.async_copy(x_hbm.at[pl.ds(base, CHUNK)], x_v.at[buf],
                         in_sems.at[buf])

    def fetch_wait(ci, buf):
        base = worker_base + ci * CHUNK
        pltpu.make_async_copy(idx_hbm.at[pl.ds(base, CHUNK)], idx_v.at[buf],
                              in_sems.at[buf]).wait()
        pltpu.make_async_copy(x_hbm.at[pl.ds(base, CHUNK)], x_v.at[buf],
                              in_sems.at[buf]).wait()

    def store_wait(ci, buf):
        base = worker_base + ci * CHUNK
        pltpu.make_async_copy(rows_v.at[buf],
                              out_hbm.at[pl.ds(base, CHUNK)],
                              out_sems.at[buf]).wait()

    fetch(0, 0)

    def chunk_body(ci, _):
        buf = lax.rem(ci, 2)
        fetch_wait(ci, buf)

        @pl.when(ci + 1 < n_chunks)
        def _prefetch():
            fetch(ci + 1, 1 - buf)

        @pl.when(ci >= 2)
        def _drain():
            store_wait(ci - 2, buf)

        def row_body(r, _):
            lane_r = jnp.broadcast_to(r, (LANES,))
            iv = plsc.load_gather(idx_v.at[buf], [lane_r])
            xs = plsc.load_gather(x_v.at[buf], [lane_r])
            for c in range(SLICES):
                tr = plsc.load_gather(tab_v, [iv, col_regs[c]])
                rows_v[buf, r, pl.ds(c * LANES, LANES)] = tr + xs * w_regs[c]
            return _

        lax.fori_loop(0, CHUNK, row_body, None)
        base = worker_base + ci * CHUNK
        pltpu.async_copy(rows_v.at[buf], out_hbm.at[pl.ds(base, CHUNK)],
                         out_sems.at[buf])
        return _

    lax.fori_loop(0, n_chunks, chunk_body, None)
    store_wait(n_chunks - 2, lax.rem(n_chunks - 2, 2))
    store_wait(n_chunks - 1, lax.rem(n_chunks - 1, 2))


def kernel(discrete_expression, normalized_expr, bin_table, W, b):
    B, G = discrete_expression.shape
    N = B * G
    idx = discrete_expression.astype(jnp.int32).reshape(N)
    x = normalized_expr.reshape(N)
    w = W[:, 0]

    mesh = plsc.VectorSubcoreMesh(core_axis_name="c", subcore_axis_name="s")
    run = pl.kernel(
        _body,
        out_type=jax.ShapeDtypeStruct((N, EMBED_DIM), jnp.float32),
        mesh=mesh,
        compiler_params=pltpu.CompilerParams(
            needs_layout_passes=False, use_tc_tiling_on_sc=False),
        scratch_types=[
            pltpu.VMEM((VOCAB, EMBED_DIM), jnp.float32),
            pltpu.VMEM((2, CHUNK), jnp.int32),
            pltpu.VMEM((2, CHUNK), jnp.float32),
            pltpu.VMEM((2, CHUNK, EMBED_DIM), jnp.float32),
            pltpu.VMEM((2, EMBED_DIM), jnp.float32),
            pltpu.SemaphoreType.DMA((2,)),
            pltpu.SemaphoreType.DMA((2,)),
        ],
    )
    out = run(idx, x, bin_table, w, b)
    return out.reshape(B, G, EMBED_DIM)


# speedup vs baseline: 3.1752x; 1.3258x over previous
import jax
import jax.numpy as jnp
from jax import lax
from jax.experimental import pallas as pl
from jax.experimental.pallas import tpu as pltpu
from jax.experimental.pallas import tpu_sc as plsc

EMBED_DIM = 64
LANES = 16
NUM_CORES = 2
NUM_SUBCORES = 16
NUM_WORKERS = NUM_CORES * NUM_SUBCORES
CHUNK = 256
SLICES = EMBED_DIM // LANES
VOCAB = 53


def _body(idx_hbm, x_hbm, tab_hbm, w_hbm, b_hbm, out_hbm,
          tab_v, w_v, b_v,
          idx_a, x_a, rows_a, in_sem_a, out_sem_a,
          idx_b, x_b, rows_b, in_sem_b, out_sem_b):
    wid = lax.axis_index("s") * NUM_CORES + lax.axis_index("c")
    rows_per_worker = out_hbm.shape[0] // NUM_WORKERS
    n_chunks = rows_per_worker // CHUNK
    worker_base = wid * rows_per_worker

    pltpu.sync_copy(w_hbm, w_v)
    pltpu.sync_copy(b_hbm, b_v)
    pltpu.sync_copy(tab_hbm, tab_v)
    w_regs = [w_v[pl.ds(c * LANES, LANES)] for c in range(SLICES)]
    b_regs = [b_v[pl.ds(c * LANES, LANES)] for c in range(SLICES)]

    def fold_row(v, _):
        for c in range(SLICES):
            sl = pl.ds(c * LANES, LANES)
            tab_v[v, sl] = tab_v[v, sl] + b_regs[c]
        return _

    lax.fori_loop(0, VOCAB, fold_row, None)

    col_regs = [c * LANES + lax.iota(jnp.int32, LANES) for c in range(SLICES)]

    def fetch(ci, idx_v, x_v, sem):
        base = worker_base + ci * CHUNK
        pltpu.async_copy(idx_hbm.at[pl.ds(base, CHUNK)], idx_v, sem)
        pltpu.async_copy(x_hbm.at[pl.ds(base, CHUNK)], x_v, sem)

    def fetch_wait(ci, idx_v, x_v, sem):
        base = worker_base + ci * CHUNK
        pltpu.make_async_copy(idx_hbm.at[pl.ds(base, CHUNK)], idx_v,
                              sem).wait()
        pltpu.make_async_copy(x_hbm.at[pl.ds(base, CHUNK)], x_v, sem).wait()

    def store_wait(ci, rows_v, sem):
        base = worker_base + ci * CHUNK
        pltpu.make_async_copy(rows_v, out_hbm.at[pl.ds(base, CHUNK)],
                              sem).wait()

    def process(ci, idx_v, x_v, rows_v, in_sem, out_sem):
        fetch_wait(ci, idx_v, x_v, in_sem)

        @pl.when(ci >= 2)
        def _drain():
            store_wait(ci - 2, rows_v, out_sem)

        def row_body(r, _):
            lane_r = jnp.broadcast_to(r, (LANES,))
            iv = plsc.load_gather(idx_v, [lane_r])
            xs = plsc.load_gather(x_v, [lane_r])
            for c in range(SLICES):
                tr = plsc.load_gather(tab_v, [iv, col_regs[c]])
                rows_v[r, pl.ds(c * LANES, LANES)] = tr + xs * w_regs[c]
            return _

        lax.fori_loop(0, CHUNK, row_body, None)
        base = worker_base + ci * CHUNK
        pltpu.async_copy(rows_v, out_hbm.at[pl.ds(base, CHUNK)], out_sem)

        @pl.when(ci + 2 < n_chunks)
        def _prefetch():
            fetch(ci + 2, idx_v, x_v, in_sem)

    fetch(0, idx_a, x_a, in_sem_a)
    fetch(1, idx_b, x_b, in_sem_b)

    def pair_body(cp, _):
        process(cp * 2, idx_a, x_a, rows_a, in_sem_a, out_sem_a)
        process(cp * 2 + 1, idx_b, x_b, rows_b, in_sem_b, out_sem_b)
        return _

    lax.fori_loop(0, n_chunks // 2, pair_body, None)
    store_wait(n_chunks - 2, rows_a, out_sem_a)
    store_wait(n_chunks - 1, rows_b, out_sem_b)


def kernel(discrete_expression, normalized_expr, bin_table, W, b):
    B, G = discrete_expression.shape
    N = B * G
    idx = discrete_expression.astype(jnp.int32).reshape(N)
    x = normalized_expr.reshape(N)
    w = W[:, 0]

    mesh = plsc.VectorSubcoreMesh(core_axis_name="c", subcore_axis_name="s")
    run = pl.kernel(
        _body,
        out_type=jax.ShapeDtypeStruct((N, EMBED_DIM), jnp.float32),
        mesh=mesh,
        compiler_params=pltpu.CompilerParams(needs_layout_passes=False),
        scratch_types=[
            pltpu.VMEM((VOCAB, EMBED_DIM), jnp.float32),
            pltpu.VMEM((EMBED_DIM,), jnp.float32),
            pltpu.VMEM((EMBED_DIM,), jnp.float32),
            pltpu.VMEM((CHUNK,), jnp.int32),
            pltpu.VMEM((CHUNK,), jnp.float32),
            pltpu.VMEM((CHUNK, EMBED_DIM), jnp.float32),
            pltpu.SemaphoreType.DMA,
            pltpu.SemaphoreType.DMA,
            pltpu.VMEM((CHUNK,), jnp.int32),
            pltpu.VMEM((CHUNK,), jnp.float32),
            pltpu.VMEM((CHUNK, EMBED_DIM), jnp.float32),
            pltpu.SemaphoreType.DMA,
            pltpu.SemaphoreType.DMA,
        ],
    )
    out = run(idx, x, bin_table, w, b)
    return out.reshape(B, G, EMBED_DIM)


# speedup vs baseline: 8.3515x; 2.6302x over previous
import jax
import jax.numpy as jnp
from jax import lax
from jax.experimental import pallas as pl
from jax.experimental.pallas import tpu as pltpu
from jax.experimental.pallas import tpu_sc as plsc

EMBED_DIM = 64
LANES = 16
NUM_CORES = 2
NUM_SUBCORES = 16
NUM_WORKERS = NUM_CORES * NUM_SUBCORES
CHUNK = 256
SLICES = EMBED_DIM // LANES
VOCAB = 53


def _body(idx_hbm, x_hbm, tab_hbm, w_hbm, b_hbm, out_hbm,
          tab_v, w_v, b_v,
          idx_a, x_a, rows_a, in_sem_a, out_sem_a,
          idx_b, x_b, rows_b, in_sem_b, out_sem_b):
    wid = lax.axis_index("s") * NUM_CORES + lax.axis_index("c")
    rows_per_worker = out_hbm.shape[0] // NUM_WORKERS
    n_chunks = rows_per_worker // CHUNK
    worker_base = wid * rows_per_worker

    pltpu.sync_copy(w_hbm, w_v)
    pltpu.sync_copy(b_hbm, b_v)
    pltpu.sync_copy(tab_hbm, tab_v)
    w_regs = [w_v[pl.ds(c * LANES, LANES)] for c in range(SLICES)]
    b_regs = [b_v[pl.ds(c * LANES, LANES)] for c in range(SLICES)]

    def fold_row(v, _):
        for c in range(SLICES):
            sl = pl.ds(c * LANES, LANES)
            tab_v[v, sl] = tab_v[v, sl] + b_regs[c]
        return _

    lax.fori_loop(0, VOCAB, fold_row, None)

    col_regs = [c * LANES + lax.iota(jnp.int32, LANES) for c in range(SLICES)]

    def fetch(ci, idx_v, x_v, sem):
        base = worker_base + ci * CHUNK
        pltpu.async_copy(idx_hbm.at[pl.ds(base, CHUNK)], idx_v, sem)
        pltpu.async_copy(x_hbm.at[pl.ds(base, CHUNK)], x_v, sem)

    def fetch_wait(ci, idx_v, x_v, sem):
        base = worker_base + ci * CHUNK
        pltpu.make_async_copy(idx_hbm.at[pl.ds(base, CHUNK)], idx_v,
                              sem).wait()
        pltpu.make_async_copy(x_hbm.at[pl.ds(base, CHUNK)], x_v, sem).wait()

    def store_wait(ci, rows_v, sem):
        base = worker_base + ci * CHUNK
        pltpu.make_async_copy(rows_v, out_hbm.at[pl.ds(base, CHUNK)],
                              sem).wait()

    def process(ci, idx_v, x_v, rows_v, in_sem, out_sem):
        fetch_wait(ci, idx_v, x_v, in_sem)

        @pl.when(ci >= 2)
        def _drain():
            store_wait(ci - 2, rows_v, out_sem)

        @plsc.parallel_loop(0, CHUNK, step=1, unroll=8)
        def row_body(r):
            lane_r = jnp.broadcast_to(r, (LANES,))
            iv = plsc.load_gather(idx_v, [lane_r])
            xs = plsc.load_gather(x_v, [lane_r])
            for c in range(SLICES):
                tr = plsc.load_gather(tab_v, [iv, col_regs[c]])
                rows_v[r, pl.ds(c * LANES, LANES)] = tr + xs * w_regs[c]
        base = worker_base + ci * CHUNK
        pltpu.async_copy(rows_v, out_hbm.at[pl.ds(base, CHUNK)], out_sem)

        @pl.when(ci + 2 < n_chunks)
        def _prefetch():
            fetch(ci + 2, idx_v, x_v, in_sem)

    fetch(0, idx_a, x_a, in_sem_a)
    fetch(1, idx_b, x_b, in_sem_b)

    def pair_body(cp, _):
        process(cp * 2, idx_a, x_a, rows_a, in_sem_a, out_sem_a)
        process(cp * 2 + 1, idx_b, x_b, rows_b, in_sem_b, out_sem_b)
        return _

    lax.fori_loop(0, n_chunks // 2, pair_body, None)
    store_wait(n_chunks - 2, rows_a, out_sem_a)
    store_wait(n_chunks - 1, rows_b, out_sem_b)


def kernel(discrete_expression, normalized_expr, bin_table, W, b):
    B, G = discrete_expression.shape
    N = B * G
    idx = discrete_expression.astype(jnp.int32).reshape(N)
    x = normalized_expr.reshape(N)
    w = W[:, 0]

    mesh = plsc.VectorSubcoreMesh(core_axis_name="c", subcore_axis_name="s")
    run = pl.kernel(
        _body,
        out_type=jax.ShapeDtypeStruct((N, EMBED_DIM), jnp.float32),
        mesh=mesh,
        compiler_params=pltpu.CompilerParams(needs_layout_passes=False),
        scratch_types=[
            pltpu.VMEM((VOCAB, EMBED_DIM), jnp.float32),
            pltpu.VMEM((EMBED_DIM,), jnp.float32),
            pltpu.VMEM((EMBED_DIM,), jnp.float32),
            pltpu.VMEM((CHUNK,), jnp.int32),
            pltpu.VMEM((CHUNK,), jnp.float32),
            pltpu.VMEM((CHUNK, EMBED_DIM), jnp.float32),
            pltpu.SemaphoreType.DMA,
            pltpu.SemaphoreType.DMA,
            pltpu.VMEM((CHUNK,), jnp.int32),
            pltpu.VMEM((CHUNK,), jnp.float32),
            pltpu.VMEM((CHUNK, EMBED_DIM), jnp.float32),
            pltpu.SemaphoreType.DMA,
            pltpu.SemaphoreType.DMA,
        ],
    )
    out = run(idx, x, bin_table, w, b)
    return out.reshape(B, G, EMBED_DIM)
